# baseline (speedup 1.0000x reference)
import functools

import jax
import jax.numpy as jnp
from jax import lax
from jax.experimental import pallas as pl
from jax.experimental.pallas import tpu as pltpu
from jax.experimental.pallas import tpu_sc as plsc

BATCH = 16384
D = 32
LANES = 16
NUM_CORES = 2
NUM_SUBCORES = 16
NUM_WORKERS = NUM_CORES * NUM_SUBCORES
B_PER_W = BATCH // NUM_WORKERS
CHUNK = 64
N_CHUNKS = B_PER_W // CHUNK
GROUPS_PER_CHUNK = CHUNK // LANES
ROW128 = 128
PACK = ROW128 // D
N_TBL_ROWS = 1000000
N_SUPER = N_TBL_ROWS // PACK


def _vad_body(idx4_hbm, col_hbm, eps_hbm, mu_hbm, lv_hbm,
              lat_out, mu_out, lv_out,
              idx_v, col_v,
              eps0, eps1, gmu0, gmu1, glv0, glv1,
              mus0, mus1, lvs0, lvs1, lats0, lats1,
              gsem, esem, osem):
    wid = lax.axis_index("s") * NUM_CORES + lax.axis_index("c")
    base = wid * B_PER_W

    pltpu.sync_copy(idx4_hbm.at[pl.ds(base, B_PER_W)], idx_v)
    pltpu.sync_copy(col_hbm.at[pl.ds(base, B_PER_W)], col_v)

    eps = (eps0, eps1)
    gmu = (gmu0, gmu1)
    glv = (glv0, glv1)
    mus = (mus0, mus1)
    lvs = (lvs0, lvs1)
    lats = (lats0, lats1)
    in_pend = [None, None]
    out_pend = [None, None]

    def issue(j):
        b = j % 2
        isl = idx_v.at[pl.ds(j * CHUNK, CHUNK)]
        in_pend[b] = (
            pltpu.async_copy(mu_hbm.at[isl], gmu[b], gsem),
            pltpu.async_copy(lv_hbm.at[isl], glv[b], gsem),
            pltpu.async_copy(eps_hbm.at[pl.ds(base + j * CHUNK, CHUNK)],
                             eps[b], esem),
        )

    issue(0)
    issue(1)
    iota = lax.iota(jnp.int32, LANES)

    for j in range(N_CHUNKS):
        b = j % 2
        for cp in in_pend[b]:
            cp.wait()
        if out_pend[b] is not None:
            for cp in out_pend[b]:
                cp.wait()
        g_mu_b, g_lv_b, eps_b = gmu[b], glv[b], eps[b]
        mus_b, lvs_b, lats_b = mus[b], lvs[b], lats[b]

        def group(g, carry, g_mu_b=g_mu_b, g_lv_b=g_lv_b, eps_b=eps_b,
                  mus_b=mus_b, lvs_b=lvs_b, lats_b=lats_b, j=j):
            rl = g * LANES + iota
            col = col_v[pl.ds(j * CHUNK + g * LANES, LANES)]
            for c in range(D):
                cc = jnp.full((LANES,), c, jnp.int32)
                m = plsc.load_gather(g_mu_b, [rl, col + c])
                v = plsc.load_gather(g_lv_b, [rl, col + c])
                e = plsc.load_gather(eps_b, [rl, cc])
                latv = m + e * jnp.exp(v * 0.5)
                plsc.store_scatter(mus_b, [rl, cc], m)
                plsc.store_scatter(lvs_b, [rl, cc], v)
                plsc.store_scatter(lats_b, [rl, cc], latv)
            return carry

        lax.fori_loop(0, GROUPS_PER_CHUNK, group, 0)

        if j + 2 < N_CHUNKS:
            issue(j + 2)

        dst = pl.ds(base + j * CHUNK, CHUNK)
        out_pend[b] = (
            pltpu.async_copy(mus_b, mu_out.at[dst], osem),
            pltpu.async_copy(lvs_b, lv_out.at[dst], osem),
            pltpu.async_copy(lats_b, lat_out.at[dst], osem),
        )

    for pend in out_pend:
        if pend is not None:
            for cp in pend:
                cp.wait()


@functools.partial(
    pl.kernel,
    out_type=(
        jax.ShapeDtypeStruct((BATCH, D), jnp.float32),
        jax.ShapeDtypeStruct((BATCH, D), jnp.float32),
        jax.ShapeDtypeStruct((BATCH, D), jnp.float32),
    ),
    mesh=plsc.VectorSubcoreMesh(core_axis_name="c", subcore_axis_name="s"),
    compiler_params=pltpu.CompilerParams(needs_layout_passes=False),
    scratch_types=[
        pltpu.VMEM((B_PER_W,), jnp.int32),
        pltpu.VMEM((B_PER_W,), jnp.int32),
        pltpu.VMEM((CHUNK, D), jnp.float32),
        pltpu.VMEM((CHUNK, D), jnp.float32),
        pltpu.VMEM((CHUNK, ROW128), jnp.float32),
        pltpu.VMEM((CHUNK, ROW128), jnp.float32),
        pltpu.VMEM((CHUNK, ROW128), jnp.float32),
        pltpu.VMEM((CHUNK, ROW128), jnp.float32),
        pltpu.VMEM((CHUNK, D), jnp.float32),
        pltpu.VMEM((CHUNK, D), jnp.float32),
        pltpu.VMEM((CHUNK, D), jnp.float32),
        pltpu.VMEM((CHUNK, D), jnp.float32),
        pltpu.VMEM((CHUNK, D), jnp.float32),
        pltpu.VMEM((CHUNK, D), jnp.float32),
        pltpu.SemaphoreType.DMA,
        pltpu.SemaphoreType.DMA,
        pltpu.SemaphoreType.DMA,
    ],
)
def _vad_kernel(idx4_hbm, col_hbm, eps_hbm, mu_hbm, lv_hbm,
                lat_out, mu_out, lv_out,
                idx_v, col_v,
                eps0, eps1, gmu0, gmu1, glv0, glv1,
                mus0, mus1, lvs0, lvs1, lats0, lats1,
                gsem, esem, osem):
    _vad_body(idx4_hbm, col_hbm, eps_hbm, mu_hbm, lv_hbm,
              lat_out, mu_out, lv_out,
              idx_v, col_v,
              eps0, eps1, gmu0, gmu1, glv0, glv1,
              mus0, mus1, lvs0, lvs1, lats0, lats1,
              gsem, esem, osem)


@jax.jit
def kernel(instance_ids, eps, weight_mu, weight_logvar):
    idx4 = lax.shift_right_logical(instance_ids, 2)
    col = lax.shift_left(jnp.bitwise_and(instance_ids, 3), 5)
    tbl_mu = weight_mu.reshape(N_SUPER, ROW128)
    tbl_lv = weight_logvar.reshape(N_SUPER, ROW128)
    lat, mu, lv = _vad_kernel(idx4, col, eps, tbl_mu, tbl_lv)
    return (lat, mu, lv)


# speedup vs baseline: 4.0430x; 4.0430x over previous
import functools

import jax
import jax.numpy as jnp
from jax import lax
from jax.experimental import pallas as pl
from jax.experimental.pallas import tpu as pltpu
from jax.experimental.pallas import tpu_sc as plsc

BATCH = 16384
D = 32
LANES = 16
NUM_CORES = 2
NUM_SUBCORES = 16
NUM_WORKERS = NUM_CORES * NUM_SUBCORES
B_PER_W = BATCH // NUM_WORKERS
TILE = 128
NBUF = 4
CHUNK = 128
N_CHUNKS = B_PER_W // CHUNK
STEPS = CHUNK // NBUF


def _vad_body(ids_hbm, epsT_hbm, muT_hbm, lvT_hbm,
              latT_out, muT_out, lvT_out,
              idx_v, eps_v, mus_v, lvs_v, lats_v,
              tmu0, tmu1, tmu2, tmu3, tlv0, tlv1, tlv2, tlv3,
              musem, lvsem, esem, osem):
    wid = lax.axis_index("s") * NUM_CORES + lax.axis_index("c")
    base = wid * B_PER_W

    pltpu.sync_copy(ids_hbm.at[pl.ds(base, B_PER_W)],
                    idx_v.at[pl.ds(0, B_PER_W)])
    eps_cp = pltpu.async_copy(epsT_hbm.at[:, pl.ds(base, B_PER_W)],
                              eps_v, esem)

    tmu = (tmu0, tmu1, tmu2, tmu3)
    tlv = (tlv0, tlv1, tlv2, tlv3)

    def issue(i, b):
        off = pl.multiple_of(lax.shift_right_logical(i, 7) * TILE, TILE)
        pltpu.async_copy(muT_hbm.at[:, pl.ds(off, TILE)], tmu[b], musem)
        pltpu.async_copy(lvT_hbm.at[:, pl.ds(off, TILE)], tlv[b], lvsem)

    ids0 = idx_v[pl.ds(0, LANES)]
    for b in range(NBUF):
        issue(ids0[b], b)

    eps_cp.wait()
    iota = lax.iota(jnp.int32, LANES)
    rows_hi = iota + LANES

    for c in range(N_CHUNKS):

        def blk(g, carry, c=c):
            for b in range(NBUF):
                p = c * CHUNK + g * NBUF + b
                pltpu.make_async_copy(
                    muT_hbm.at[:, pl.ds(0, TILE)], tmu[b], musem).wait()
                pltpu.make_async_copy(
                    lvT_hbm.at[:, pl.ds(0, TILE)], tlv[b], lvsem).wait()

                i = idx_v[pl.ds(p, LANES)][0]
                lane = jnp.full((LANES,), i & (TILE - 1), jnp.int32)
                colp = jnp.full((LANES,), g * NBUF + b, jnp.int32)
                pcol = jnp.full((LANES,), p, jnp.int32)

                m_lo = plsc.load_gather(tmu[b], [iota, lane])
                m_hi = plsc.load_gather(tmu[b], [rows_hi, lane])
                v_lo = plsc.load_gather(tlv[b], [iota, lane])
                v_hi = plsc.load_gather(tlv[b], [rows_hi, lane])
                e_lo = plsc.load_gather(eps_v, [iota, pcol])
                e_hi = plsc.load_gather(eps_v, [rows_hi, pcol])
                l_lo = m_lo + e_lo * jnp.exp(v_lo * 0.5)
                l_hi = m_hi + e_hi * jnp.exp(v_hi * 0.5)

                plsc.store_scatter(mus_v, [iota, colp], m_lo)
                plsc.store_scatter(mus_v, [rows_hi, colp], m_hi)
                plsc.store_scatter(lvs_v, [iota, colp], v_lo)
                plsc.store_scatter(lvs_v, [rows_hi, colp], v_hi)
                plsc.store_scatter(lats_v, [iota, colp], l_lo)
                plsc.store_scatter(lats_v, [rows_hi, colp], l_hi)

                @pl.when(p + NBUF < B_PER_W)
                def _():
                    iq = idx_v[pl.ds(p + NBUF, LANES)][0]
                    issue(iq, b)

            return carry

        lax.fori_loop(0, STEPS, blk, 0)

        dst = pl.ds(base + c * CHUNK, CHUNK)
        cps = (
            pltpu.async_copy(mus_v, muT_out.at[:, dst], osem),
            pltpu.async_copy(lvs_v, lvT_out.at[:, dst], osem),
            pltpu.async_copy(lats_v, latT_out.at[:, dst], osem),
        )
        for cp in cps:
            cp.wait()


@functools.partial(
    pl.kernel,
    out_type=(
        jax.ShapeDtypeStruct((D, BATCH), jnp.float32),
        jax.ShapeDtypeStruct((D, BATCH), jnp.float32),
        jax.ShapeDtypeStruct((D, BATCH), jnp.float32),
    ),
    mesh=plsc.VectorSubcoreMesh(core_axis_name="c", subcore_axis_name="s"),
    compiler_params=pltpu.CompilerParams(needs_layout_passes=False),
    scratch_types=[
        pltpu.VMEM((B_PER_W + 2 * LANES,), jnp.int32),
        pltpu.VMEM((D, B_PER_W), jnp.float32),
        pltpu.VMEM((D, CHUNK), jnp.float32),
        pltpu.VMEM((D, CHUNK), jnp.float32),
        pltpu.VMEM((D, CHUNK), jnp.float32),
        pltpu.VMEM((D, TILE), jnp.float32),
        pltpu.VMEM((D, TILE), jnp.float32),
        pltpu.VMEM((D, TILE), jnp.float32),
        pltpu.VMEM((D, TILE), jnp.float32),
        pltpu.VMEM((D, TILE), jnp.float32),
        pltpu.VMEM((D, TILE), jnp.float32),
        pltpu.VMEM((D, TILE), jnp.float32),
        pltpu.VMEM((D, TILE), jnp.float32),
        pltpu.SemaphoreType.DMA,
        pltpu.SemaphoreType.DMA,
        pltpu.SemaphoreType.DMA,
        pltpu.SemaphoreType.DMA,
    ],
)
def _vad_kernel(ids_hbm, epsT_hbm, muT_hbm, lvT_hbm,
                latT_out, muT_out, lvT_out,
                idx_v, eps_v, mus_v, lvs_v, lats_v,
                tmu0, tmu1, tmu2, tmu3, tlv0, tlv1, tlv2, tlv3,
                musem, lvsem, esem, osem):
    _vad_body(ids_hbm, epsT_hbm, muT_hbm, lvT_hbm,
              latT_out, muT_out, lvT_out,
              idx_v, eps_v, mus_v, lvs_v, lats_v,
              tmu0, tmu1, tmu2, tmu3, tlv0, tlv1, tlv2, tlv3,
              musem, lvsem, esem, osem)


@jax.jit
def kernel(instance_ids, eps, weight_mu, weight_logvar):
    latT, muT, lvT = _vad_kernel(
        instance_ids, eps.T, weight_mu.T, weight_logvar.T)
    return (latT.T, muT.T, lvT.T)
